# baseline (speedup 1.0000x reference)
import jax
import jax.numpy as jnp
from jax.experimental import pallas as pl
from jax.experimental.pallas import tpu as pltpu


def _body(t_ref, x_ref, out_ref):
    i = pl.program_id(0)
    t_row = t_ref[pl.ds(i, 1), :]
    out_ref[...] = t_row[:, None, :] + x_ref[...][None, :, :]


def kernel(T, n_x, t_embed, x_embed):
    max_t, d = t_embed.shape
    max_x = x_embed.shape[0]
    out = pl.pallas_call(
        _body,
        grid=(max_t,),
        in_specs=[
            pl.BlockSpec((max_t, d), lambda i: (0, 0)),
            pl.BlockSpec((max_x, d), lambda i: (0, 0)),
        ],
        out_specs=pl.BlockSpec((1, max_x, d), lambda i: (i, 0, 0)),
        out_shape=jax.ShapeDtypeStruct((max_t, max_x, d), jnp.float32),
        compiler_params=pltpu.CompilerParams(
            dimension_semantics=("arbitrary",),
        ),
    )(t_embed, x_embed)
    return out
